# baseline (speedup 1.0000x reference)
import jax
import jax.numpy as jnp
from jax.experimental import pallas as pl

_N = 26


def _plane_body(idxt_ref, o_ref):
    k = pl.program_id(1)
    o_ref[...] = (idxt_ref[...] == k).astype(jnp.int32)[None]


def kernel(idxs_vec):
    b, l = idxs_vec.shape
    h = l // 5
    idxt = idxs_vec.T
    out3 = pl.pallas_call(
        _plane_body,
        grid=(5, _N),
        in_specs=[pl.BlockSpec((h, b), lambda c, k: (c, 0))],
        out_specs=pl.BlockSpec((1, h, b), lambda c, k: (k, c, 0)),
        out_shape=jax.ShapeDtypeStruct((_N, l, b), jnp.int32),
    )(idxt)
    return jnp.transpose(out3, (2, 1, 0))


# speedup vs baseline: 2.2526x; 2.2526x over previous
import jax
import jax.numpy as jnp
from jax.experimental import pallas as pl

_N = 26


def _plane_body(idxt_ref, o_ref):
    k = pl.program_id(0)
    o_ref[...] = (idxt_ref[...] == k).astype(jnp.int32)[None]


def kernel(idxs_vec):
    b, l = idxs_vec.shape
    idxt = idxs_vec.T
    out3 = pl.pallas_call(
        _plane_body,
        grid=(_N,),
        in_specs=[pl.BlockSpec((l, b), lambda k: (0, 0))],
        out_specs=pl.BlockSpec((1, l, b), lambda k: (k, 0, 0)),
        out_shape=jax.ShapeDtypeStruct((_N, l, b), jnp.int32),
    )(idxt)
    return jnp.transpose(out3, (2, 1, 0))
